# Initial kernel scaffold; baseline (speedup 1.0000x reference)
import jax
import jax.numpy as jnp
from jax.experimental import pallas as pl


def kernel(pos_u, pos_w, neg_w, W_word, W_ctx):
    raise NotImplementedError("write your pallas kernel here")


# speedup vs baseline: 1.2440x; 1.2440x over previous
import functools

import jax
import jax.numpy as jnp
from jax import lax
from jax.experimental import pallas as pl
from jax.experimental.pallas import tpu as pltpu
from jax.experimental.pallas import tpu_sc as plsc

V = 100000
D = 128
D2 = 2 * D
B = 16384
NNEG = 20
NC, NS = 2, 16
NW = NC * NS
BPW = B // NW
C = 16
CN = C * NNEG
NCHUNK = BPW // C

_PC = (
    -0.00012003165279301213,
    0.00022841986642438663,
    0.0023642433409658313,
    -0.0049591490971462471,
    -0.079832648991337157,
    0.72134636444934586,
    -0.3665129644162643,
)


def _logsp(t):
    r = t * _PC[0] + _PC[1]
    for c in _PC[2:]:
        r = r * t + c
    return r


_TX_ROWS = 1000


def _tx_one(vec):
    w = vec[:, :D]
    Wc = vec[:, D:]
    z = jax.nn.sigmoid(w)
    Z = z + jax.nn.sigmoid(Wc) * (1.0 - z)
    return jnp.concatenate([z, Z], axis=1)


def _tx_kernel(word_ref, ctx_ref, ow_ref, oc_ref):
    ow_ref[...] = _tx_one(word_ref[...])
    oc_ref[...] = _tx_one(ctx_ref[...])


def _transform_tables(W_word, W_ctx):
    spec = pl.BlockSpec((_TX_ROWS, D2), lambda i: (i, 0))
    return pl.pallas_call(
        _tx_kernel,
        grid=(V // _TX_ROWS,),
        in_specs=[spec, spec],
        out_specs=[spec, spec],
        out_shape=[jax.ShapeDtypeStruct((V, D2), jnp.float32)] * 2,
    )(W_word, W_ctx)


_sc_mesh = plsc.VectorSubcoreMesh(core_axis_name="c", subcore_axis_name="s")


def _lanesum(x):
    lanes = lax.iota(jnp.int32, 16)
    dnums = lax.GatherDimensionNumbers(
        offset_dims=(), collapsed_slice_dims=(0,), start_index_map=(0,))
    for s in (8, 4, 2, 1):
        idx = jnp.bitwise_xor(lanes, s)
        x = x + lax.gather(
            x, idx[:, None], dnums, slice_sizes=(1,),
            mode=lax.GatherScatterMode.PROMISE_IN_BOUNDS)
    return x


def _set_lane(vec, lane_idx, val_vec):
    lanes = lax.iota(jnp.int32, 16)
    return jnp.where(lanes == lane_idx, val_vec, vec)


@functools.partial(
    pl.kernel,
    out_type=[
        jax.ShapeDtypeStruct((B,), jnp.float32),
        jax.ShapeDtypeStruct((B,), jnp.float32),
        jax.ShapeDtypeStruct((B * NNEG,), jnp.float32),
        jax.ShapeDtypeStruct((B,), jnp.float32),
        jax.ShapeDtypeStruct((B * NNEG,), jnp.float32),
    ],
    mesh=_sc_mesh,
    scratch_types=[
        pltpu.VMEM((BPW,), jnp.int32),
        pltpu.VMEM((BPW,), jnp.int32),
        pltpu.VMEM((BPW * NNEG,), jnp.int32),
        pltpu.VMEM((C, D2), jnp.float32),
        pltpu.VMEM((C, D2), jnp.float32),
        pltpu.VMEM((CN, D2), jnp.float32),
        pltpu.VMEM((BPW,), jnp.float32),
        pltpu.VMEM((BPW,), jnp.float32),
        pltpu.VMEM((BPW * NNEG,), jnp.float32),
        pltpu.VMEM((BPW,), jnp.float32),
        pltpu.VMEM((BPW * NNEG,), jnp.float32),
        pltpu.SemaphoreType.DMA,
    ],
)
def _sc_volumes(pos_u_h, pos_w_h, negf_h, zzw_h, zzc_h,
                tv_h, pv_h, nv_h, pi_h, ni_h,
                idx_u, idx_w, idx_n, rows_u, rows_w, rows_n,
                o_tv, o_pv, o_nv, o_pi, o_ni, sem):
    wid = lax.axis_index("c") * NS + lax.axis_index("s")
    base = wid * BPW

    pltpu.sync_copy(pos_u_h.at[pl.ds(base, BPW)], idx_u)
    pltpu.sync_copy(pos_w_h.at[pl.ds(base, BPW)], idx_w)
    pltpu.sync_copy(negf_h.at[pl.ds(base * NNEG, BPW * NNEG)], idx_n)

    zero16 = jnp.zeros((16,), jnp.float32)

    def chunk_body(ci, _):
        off = pl.multiple_of(ci * C, 8)
        noff = pl.multiple_of(ci * CN, 8)
        cp1 = pltpu.async_copy(zzw_h.at[idx_u.at[pl.ds(off, C)]], rows_u, sem)
        cp2 = pltpu.async_copy(zzc_h.at[idx_w.at[pl.ds(off, C)]], rows_w, sem)
        cp3 = pltpu.async_copy(
            zzc_h.at[idx_n.at[pl.ds(noff, 128)]],
            rows_n.at[pl.ds(0, 128)], sem)
        cp4 = pltpu.async_copy(
            zzc_h.at[idx_n.at[pl.ds(noff + 128, 128)]],
            rows_n.at[pl.ds(128, 128)], sem)
        cp5 = pltpu.async_copy(
            zzc_h.at[idx_n.at[pl.ds(noff + 256, CN - 256)]],
            rows_n.at[pl.ds(256, CN - 256)], sem)
        cp1.wait()
        cp2.wait()
        cp3.wait()
        cp4.wait()
        cp5.wait()

        def elem_body(k, carry):
            cur_tv, cur_pv, cur_pi, cur_nv, cur_ni = carry
            zu = [rows_u[k, pl.ds(d * 16, 16)] for d in range(D // 16)]
            Zu = [rows_u[k, pl.ds(D + d * 16, 16)] for d in range(D // 16)]
            acc_t = _logsp(Zu[0] - zu[0])
            for d in range(1, D // 16):
                acc_t = acc_t + _logsp(Zu[d] - zu[d])
            acc_p = zero16
            acc_i = zero16
            for d in range(D // 16):
                zw = rows_w[k, pl.ds(d * 16, 16)]
                Zw = rows_w[k, pl.ds(D + d * 16, 16)]
                acc_p = acc_p + _logsp(Zw - zw)
                acc_i = acc_i + _logsp(
                    jnp.minimum(Zw, Zu[d]) - jnp.maximum(zw, zu[d]))
            cur_tv = _set_lane(cur_tv, k, _lanesum(acc_t))
            cur_pv = _set_lane(cur_pv, k, _lanesum(acc_p))
            cur_pi = _set_lane(cur_pi, k, _lanesum(acc_i))

            @pl.when(k == C - 1)
            def _():
                o_tv[pl.ds(pl.multiple_of(ci * C, 8), 16)] = cur_tv
                o_pv[pl.ds(pl.multiple_of(ci * C, 8), 16)] = cur_pv
                o_pi[pl.ds(pl.multiple_of(ci * C, 8), 16)] = cur_pi

            def neg_body(j, ncarry):
                cnv, cni = ncarry
                r = k * NNEG + j
                accb = zero16
                acci = zero16
                for d in range(D // 16):
                    zn = rows_n[r, pl.ds(d * 16, 16)]
                    Zn = rows_n[r, pl.ds(D + d * 16, 16)]
                    accb = accb + _logsp(Zn - zn)
                    acci = acci + _logsp(
                        jnp.minimum(Zn, Zu[d]) - jnp.maximum(zn, zu[d]))
                lane = lax.rem(r, 16)
                cnv = _set_lane(cnv, lane, _lanesum(accb))
                cni = _set_lane(cni, lane, _lanesum(acci))

                @pl.when(lane == 15)
                def _():
                    foff = pl.multiple_of(ci * CN + (r - 15), 8)
                    o_nv[pl.ds(foff, 16)] = cnv
                    o_ni[pl.ds(foff, 16)] = cni

                return (cnv, cni)

            cur_nv, cur_ni = lax.fori_loop(
                0, NNEG, neg_body, (cur_nv, cur_ni))
            return (cur_tv, cur_pv, cur_pi, cur_nv, cur_ni)

        lax.fori_loop(0, C, elem_body,
                      (zero16, zero16, zero16, zero16, zero16))
        return 0

    lax.fori_loop(0, NCHUNK, chunk_body, 0)

    pltpu.sync_copy(o_tv, tv_h.at[pl.ds(base, BPW)])
    pltpu.sync_copy(o_pv, pv_h.at[pl.ds(base, BPW)])
    pltpu.sync_copy(o_nv, nv_h.at[pl.ds(base * NNEG, BPW * NNEG)])
    pltpu.sync_copy(o_pi, pi_h.at[pl.ds(base, BPW)])
    pltpu.sync_copy(o_ni, ni_h.at[pl.ds(base * NNEG, BPW * NNEG)])


def kernel(pos_u, pos_w, neg_w, W_word, W_ctx):
    zzw, zzc = _transform_tables(W_word, W_ctx)
    neg_flat = neg_w.reshape(-1)
    tv, pv, nvf, pi, nif = _sc_volumes(pos_u, pos_w, neg_flat, zzw, zzc)
    return (tv, pv, nvf.reshape(B, NNEG), pi, nif.reshape(B, NNEG))
